# Initial kernel scaffold; baseline (speedup 1.0000x reference)
import jax
import jax.numpy as jnp
from jax.experimental import pallas as pl


def kernel(x, edge_index, batch, W1a, b1a, W2a, b2a, W1b, b1b, W2b, b2b, Wc, bc):
    raise NotImplementedError("write your pallas kernel here")


# speedup vs baseline: 3.2327x; 3.2327x over previous
import functools

import jax
import jax.numpy as jnp
from jax import lax
from jax.experimental import pallas as pl
from jax.experimental.pallas import tpu as pltpu
from jax.experimental.pallas import tpu_sc as plsc

_NC = 2
_NS = 16
_ECH = 80
_PCH = 128
_G = 64
_GP = 72


def _make_segsum(NP2, E, W, npass):
    NPH = NP2 // _NC
    NACC = NPH + 8 * _NS
    rz = NACC // _NS
    ro = NPH // _NS
    nch = E // (_NS * _ECH)
    mesh = plsc.VectorSubcoreMesh(core_axis_name="c", subcore_axis_name="s",
                                  num_cores=_NC, num_subcores=_NS)

    @functools.partial(
        pl.kernel,
        out_type=jax.ShapeDtypeStruct((npass * NP2, W), jnp.float32),
        mesh=mesh,
        scratch_types=[
            pltpu.VMEM((nch, _ECH), jnp.int32),
            pltpu.VMEM((nch, _ECH), jnp.int32),
            pltpu.VMEM((_ECH, W), jnp.float32),
            pltpu.VMEM_SHARED((NACC, W), jnp.float32),
            pltpu.SemaphoreType.DMA,
        ],
    )
    def segsum(table, srcs, dsts, zz, out, idxs_v, idxd_v, rows_v, acc, gsem):
        cid = lax.axis_index("c")
        sid = lax.axis_index("s")
        pltpu.sync_copy(dsts.at[cid, sid], idxd_v)
        for p in range(npass):
            pltpu.sync_copy(zz.at[pl.ds(sid * rz, rz)], acc.at[pl.ds(sid * rz, rz)])
            pltpu.sync_copy(srcs.at[p, sid], idxs_v)
            plsc.subcore_barrier()

            def body(i, carry):
                pltpu.async_copy(table.at[idxs_v.at[i]], rows_v, gsem).wait()
                pltpu.sync_copy(rows_v, acc.at[idxd_v.at[i]], add=True)
                return carry

            lax.fori_loop(0, nch, body, 0)
            plsc.subcore_barrier()
            pltpu.sync_copy(acc.at[pl.ds(sid * ro, ro)],
                            out.at[pl.ds(p * NP2 + cid * NPH + sid * ro, ro)])
            if p + 1 < npass:
                plsc.subcore_barrier()

    return segsum


def _make_pool(N, D2):
    npch = N // (_NS * _PCH)
    rpt = N // _NS
    mesh = plsc.VectorSubcoreMesh(core_axis_name="c", subcore_axis_name="s",
                                  num_cores=_NC, num_subcores=_NS)

    @functools.partial(
        pl.kernel,
        out_type=(jax.ShapeDtypeStruct((_NC, _GP, D2), jnp.float32),
                  jax.ShapeDtypeStruct((_GP, D2), jnp.float32)),
        mesh=mesh,
        scratch_types=[
            pltpu.VMEM((npch, _PCH), jnp.int32),
            pltpu.VMEM((_PCH, D2), jnp.float32),
            pltpu.VMEM((_PCH, D2), jnp.float32),
            pltpu.VMEM_SHARED((_GP, D2), jnp.float32),
            pltpu.VMEM_SHARED((_GP, D2), jnp.float32),
            pltpu.SemaphoreType.DMA,
        ],
    )
    def pool(h2s, batch3, zgd, onesd, sums_out, cnt_out,
             bidx_v, rows_v, ones_v, accp, accc, sem):
        cid = lax.axis_index("c")
        sid = lax.axis_index("s")

        @pl.when(sid == 0)
        def _zero():
            pltpu.sync_copy(zgd, accp)
            pltpu.sync_copy(zgd, accc)

        pltpu.sync_copy(batch3.at[sid], bidx_v)
        pltpu.sync_copy(onesd, ones_v)
        plsc.subcore_barrier()

        def body(i, carry):
            r0 = cid * N + sid * rpt + i * _PCH
            pltpu.async_copy(h2s.at[pl.ds(r0, _PCH)], rows_v, sem).wait()
            pltpu.sync_copy(rows_v, accp.at[bidx_v.at[i]], add=True)

            @pl.when(cid == 0)
            def _cnt():
                pltpu.sync_copy(ones_v, accc.at[bidx_v.at[i]], add=True)

            return carry

        lax.fori_loop(0, npch, body, 0)
        plsc.subcore_barrier()

        @pl.when(sid == 0)
        def _out():
            pltpu.sync_copy(accp, sums_out.at[cid])

        @pl.when((sid == 0) & (cid == 0))
        def _outc():
            pltpu.sync_copy(accc, cnt_out)

    return pool


def _mlp1_body(x_ref, a_ref, W1_ref, b1_ref, W2_ref, b2_ref, out_ref):
    h = x_ref[...] + a_ref[...]
    h = jnp.maximum(jnp.dot(h, W1_ref[...], preferred_element_type=jnp.float32)
                    + b1_ref[...], 0.0)
    y = jnp.maximum(jnp.dot(h, W2_ref[...], preferred_element_type=jnp.float32)
                    + b2_ref[...], 0.0)
    half = y.shape[1] // 2
    out_ref[0] = y[:, :half]
    out_ref[1] = y[:, half:]


def _mlp2_body(h_ref, a_ref, W1_ref, b1_ref, W2_ref, b2_ref, out_ref):
    h = jnp.concatenate([h_ref[0] + a_ref[0], h_ref[1] + a_ref[1]], axis=1)
    h = jnp.maximum(jnp.dot(h, W1_ref[...], preferred_element_type=jnp.float32)
                    + b1_ref[...], 0.0)
    y = jnp.maximum(jnp.dot(h, W2_ref[...], preferred_element_type=jnp.float32)
                    + b2_ref[...], 0.0)
    half = y.shape[1] // 2
    out_ref[0] = y[:, :half]
    out_ref[1] = y[:, half:]


def _final_body(s_ref, c_ref, Wc_ref, bc_ref, out_ref):
    sums = jnp.concatenate([s_ref[0, :_G], s_ref[1, :_G]], axis=1)
    cnt = c_ref[:_G, 0:1]
    mean = sums / jnp.maximum(cnt, 1.0)
    logits = jnp.dot(mean, Wc_ref[...], preferred_element_type=jnp.float32) + bc_ref[...]
    m = jnp.max(logits, axis=1, keepdims=True)
    lse = jnp.log(jnp.sum(jnp.exp(logits - m), axis=1, keepdims=True)) + m
    out_ref[...] = logits - lse


def kernel(x, edge_index, batch, W1a, b1a, W2a, b2a, W1b, b1b, W2b, b2b, Wc, bc):
    N, D = x.shape
    E = edge_index.shape[1]
    H = W1a.shape[1]
    C = Wc.shape[1]
    D2a, D2b = D // 2, H // 2
    NP = ((N + _NS * _PCH - 1) // (_NS * _PCH)) * (_NS * _PCH)
    R = 1024

    src, dst = edge_index[0], edge_index[1]
    nch = E // (_NS * _ECH)
    NPH = NP // _NC
    NACC = NPH + 8 * _NS
    src3 = src.reshape(_NS, nch, _ECH)
    dsts = jnp.stack([jnp.where(dst < NPH, dst, NPH),
                      jnp.where(dst >= NPH, dst - NPH, NPH)])
    dsts = dsts.reshape(_NC, _NS, nch, _ECH)
    zacc = jnp.zeros((NACC, D2b), jnp.float32)

    xp = jnp.pad(x, ((0, NP - N), (0, 0)))
    agg1 = _make_segsum(NP, E, D, 1)(xp, src3[None], dsts, zacc)

    hs = pl.pallas_call(
        _mlp1_body,
        grid=(NP // R,),
        in_specs=[
            pl.BlockSpec((R, D), lambda i: (i, 0)),
            pl.BlockSpec((R, D), lambda i: (i, 0)),
            pl.BlockSpec((D, H), lambda i: (0, 0)),
            pl.BlockSpec((1, H), lambda i: (0, 0)),
            pl.BlockSpec((H, H), lambda i: (0, 0)),
            pl.BlockSpec((1, H), lambda i: (0, 0)),
        ],
        out_specs=pl.BlockSpec((_NC, R, D2b), lambda i: (0, i, 0)),
        out_shape=jax.ShapeDtypeStruct((_NC, NP, D2b), jnp.float32),
    )(xp, agg1, W1a, b1a.reshape(1, H), W2a, b2a.reshape(1, H))

    srcs2 = jnp.stack([src3, src3 + NP])
    agg2 = _make_segsum(NP, E, D2b, 2)(
        hs.reshape(_NC * NP, D2b), srcs2, dsts, zacc).reshape(_NC, NP, D2b)

    h2s = pl.pallas_call(
        _mlp2_body,
        grid=(NP // R,),
        in_specs=[
            pl.BlockSpec((_NC, R, D2b), lambda i: (0, i, 0)),
            pl.BlockSpec((_NC, R, D2b), lambda i: (0, i, 0)),
            pl.BlockSpec((H, H), lambda i: (0, 0)),
            pl.BlockSpec((1, H), lambda i: (0, 0)),
            pl.BlockSpec((H, H), lambda i: (0, 0)),
            pl.BlockSpec((1, H), lambda i: (0, 0)),
        ],
        out_specs=pl.BlockSpec((_NC, R, D2b), lambda i: (0, i, 0)),
        out_shape=jax.ShapeDtypeStruct((_NC, NP, D2b), jnp.float32),
    )(hs, agg2, W1b, b1b.reshape(1, H), W2b, b2b.reshape(1, H))

    batchp = jnp.pad(batch, (0, NP - N), constant_values=_G)
    batch3 = batchp.reshape(_NS, NP // (_NS * _PCH), _PCH)
    zgd = jnp.zeros((_GP, D2b), jnp.float32)
    onesd = jnp.ones((_PCH, D2b), jnp.float32)
    sums, cnt = _make_pool(NP, D2b)(h2s.reshape(_NC * NP, D2b), batch3,
                                    zgd, onesd)

    out = pl.pallas_call(
        _final_body,
        in_specs=[
            pl.BlockSpec((_NC, _GP, D2b), lambda: (0, 0, 0)),
            pl.BlockSpec((_GP, D2b), lambda: (0, 0)),
            pl.BlockSpec((H, C), lambda: (0, 0)),
            pl.BlockSpec((1, C), lambda: (0, 0)),
        ],
        out_specs=pl.BlockSpec((_G, C), lambda: (0, 0)),
        out_shape=jax.ShapeDtypeStruct((_G, C), jnp.float32),
    )(sums, cnt, Wc, bc.reshape(1, C))
    return out


# speedup vs baseline: 4.8141x; 1.4892x over previous
import functools

import jax
import jax.numpy as jnp
from jax import lax
from jax.experimental import pallas as pl
from jax.experimental.pallas import tpu as pltpu
from jax.experimental.pallas import tpu_sc as plsc

_NC = 2
_NS = 16
_ECH = 80
_PCH = 128
_G = 64
_GP = 72


def _make_segsum(NP2, E, W, npass):
    NPH = NP2 // _NC
    NACC = NPH + 8 * _NS
    rz = NACC // _NS
    ro = NPH // _NS
    nch = E // (_NS * _ECH)
    mesh = plsc.VectorSubcoreMesh(core_axis_name="c", subcore_axis_name="s",
                                  num_cores=_NC, num_subcores=_NS)

    @functools.partial(
        pl.kernel,
        out_type=jax.ShapeDtypeStruct((npass * NP2, W), jnp.float32),
        mesh=mesh,
        scratch_types=[
            pltpu.VMEM((nch, _ECH), jnp.int32),
            pltpu.VMEM((nch, _ECH), jnp.int32),
            pltpu.VMEM((_ECH, W), jnp.float32),
            pltpu.VMEM((_ECH, W), jnp.float32),
            pltpu.VMEM_SHARED((NACC, W), jnp.float32),
            pltpu.SemaphoreType.DMA,
            pltpu.SemaphoreType.DMA,
        ],
    )
    def segsum(table, srcs, dsts, zz, out, idxs_v, idxd_v, rows0, rows1,
               acc, g0, g1):
        cid = lax.axis_index("c")
        sid = lax.axis_index("s")
        pltpu.sync_copy(dsts.at[cid, sid], idxd_v)
        for p in range(npass):
            pltpu.sync_copy(zz.at[pl.ds(sid * rz, rz)], acc.at[pl.ds(sid * rz, rz)])
            pltpu.sync_copy(srcs.at[p, sid], idxs_v)
            plsc.subcore_barrier()
            pltpu.async_copy(table.at[idxs_v.at[0]], rows0, g0)

            def body(j, carry):
                i0 = 2 * j
                i1 = i0 + 1
                pltpu.async_copy(table.at[idxs_v.at[i1]], rows1, g1)
                pltpu.make_async_copy(table.at[idxs_v.at[i0]], rows0, g0).wait()
                pltpu.sync_copy(rows0, acc.at[idxd_v.at[i0]], add=True)

                @pl.when(i1 + 1 < nch)
                def _prefetch():
                    pltpu.async_copy(table.at[idxs_v.at[i1 + 1]], rows0, g0)

                pltpu.make_async_copy(table.at[idxs_v.at[i1]], rows1, g1).wait()
                pltpu.sync_copy(rows1, acc.at[idxd_v.at[i1]], add=True)
                return carry

            lax.fori_loop(0, nch // 2, body, 0)
            plsc.subcore_barrier()
            pltpu.sync_copy(acc.at[pl.ds(sid * ro, ro)],
                            out.at[pl.ds(p * NP2 + cid * NPH + sid * ro, ro)])
            if p + 1 < npass:
                plsc.subcore_barrier()

    return segsum


def _make_pool(N, D2):
    npch = N // (_NS * _PCH)
    rpt = N // _NS
    mesh = plsc.VectorSubcoreMesh(core_axis_name="c", subcore_axis_name="s",
                                  num_cores=_NC, num_subcores=_NS)

    @functools.partial(
        pl.kernel,
        out_type=(jax.ShapeDtypeStruct((_NC, _GP, D2), jnp.float32),
                  jax.ShapeDtypeStruct((_GP, D2), jnp.float32)),
        mesh=mesh,
        scratch_types=[
            pltpu.VMEM((npch, _PCH), jnp.int32),
            pltpu.VMEM((_PCH, D2), jnp.float32),
            pltpu.VMEM((_PCH, D2), jnp.float32),
            pltpu.VMEM_SHARED((_GP, D2), jnp.float32),
            pltpu.VMEM_SHARED((_GP, D2), jnp.float32),
            pltpu.SemaphoreType.DMA,
        ],
    )
    def pool(h2s, batch3, zgd, onesd, sums_out, cnt_out,
             bidx_v, rows_v, ones_v, accp, accc, sem):
        cid = lax.axis_index("c")
        sid = lax.axis_index("s")

        @pl.when(sid == 0)
        def _zero():
            pltpu.sync_copy(zgd, accp)
            pltpu.sync_copy(zgd, accc)

        pltpu.sync_copy(batch3.at[sid], bidx_v)
        pltpu.sync_copy(onesd, ones_v)
        plsc.subcore_barrier()

        def body(i, carry):
            r0 = cid * N + sid * rpt + i * _PCH
            pltpu.async_copy(h2s.at[pl.ds(r0, _PCH)], rows_v, sem).wait()
            pltpu.sync_copy(rows_v, accp.at[bidx_v.at[i]], add=True)

            @pl.when(cid == 0)
            def _cnt():
                pltpu.sync_copy(ones_v, accc.at[bidx_v.at[i]], add=True)

            return carry

        lax.fori_loop(0, npch, body, 0)
        plsc.subcore_barrier()

        @pl.when(sid == 0)
        def _out():
            pltpu.sync_copy(accp, sums_out.at[cid])

        @pl.when((sid == 0) & (cid == 0))
        def _outc():
            pltpu.sync_copy(accc, cnt_out)

    return pool


def _mlp1_body(x_ref, a_ref, W1_ref, b1_ref, W2_ref, b2_ref, out_ref):
    h = x_ref[...] + a_ref[...]
    h = jnp.maximum(jnp.dot(h, W1_ref[...], preferred_element_type=jnp.float32)
                    + b1_ref[...], 0.0)
    y = jnp.maximum(jnp.dot(h, W2_ref[...], preferred_element_type=jnp.float32)
                    + b2_ref[...], 0.0)
    half = y.shape[1] // 2
    out_ref[0] = y[:, :half]
    out_ref[1] = y[:, half:]


def _mlp2_body(h_ref, a_ref, W1_ref, b1_ref, W2_ref, b2_ref, out_ref):
    h = jnp.concatenate([h_ref[0] + a_ref[0], h_ref[1] + a_ref[1]], axis=1)
    h = jnp.maximum(jnp.dot(h, W1_ref[...], preferred_element_type=jnp.float32)
                    + b1_ref[...], 0.0)
    y = jnp.maximum(jnp.dot(h, W2_ref[...], preferred_element_type=jnp.float32)
                    + b2_ref[...], 0.0)
    half = y.shape[1] // 2
    out_ref[0] = y[:, :half]
    out_ref[1] = y[:, half:]


def _final_body(s_ref, c_ref, Wc_ref, bc_ref, out_ref):
    sums = jnp.concatenate([s_ref[0, :_G], s_ref[1, :_G]], axis=1)
    cnt = c_ref[:_G, 0:1]
    mean = sums / jnp.maximum(cnt, 1.0)
    logits = jnp.dot(mean, Wc_ref[...], preferred_element_type=jnp.float32) + bc_ref[...]
    m = jnp.max(logits, axis=1, keepdims=True)
    lse = jnp.log(jnp.sum(jnp.exp(logits - m), axis=1, keepdims=True)) + m
    out_ref[...] = logits - lse


def kernel(x, edge_index, batch, W1a, b1a, W2a, b2a, W1b, b1b, W2b, b2b, Wc, bc):
    N, D = x.shape
    E = edge_index.shape[1]
    H = W1a.shape[1]
    C = Wc.shape[1]
    D2a, D2b = D // 2, H // 2
    NP = ((N + _NS * _PCH - 1) // (_NS * _PCH)) * (_NS * _PCH)
    R = 1024

    src, dst = edge_index[0], edge_index[1]
    nch = E // (_NS * _ECH)
    NPH = NP // _NC
    NACC = NPH + 8 * _NS
    src3 = src.reshape(_NS, nch, _ECH)
    dsts = jnp.stack([jnp.where(dst < NPH, dst, NPH),
                      jnp.where(dst >= NPH, dst - NPH, NPH)])
    dsts = dsts.reshape(_NC, _NS, nch, _ECH)
    zacc = jnp.zeros((NACC, D2b), jnp.float32)

    xp = jnp.pad(x, ((0, NP - N), (0, 0)))
    agg1 = _make_segsum(NP, E, D, 1)(xp, src3[None], dsts, zacc)

    hs = pl.pallas_call(
        _mlp1_body,
        grid=(NP // R,),
        in_specs=[
            pl.BlockSpec((R, D), lambda i: (i, 0)),
            pl.BlockSpec((R, D), lambda i: (i, 0)),
            pl.BlockSpec((D, H), lambda i: (0, 0)),
            pl.BlockSpec((1, H), lambda i: (0, 0)),
            pl.BlockSpec((H, H), lambda i: (0, 0)),
            pl.BlockSpec((1, H), lambda i: (0, 0)),
        ],
        out_specs=pl.BlockSpec((_NC, R, D2b), lambda i: (0, i, 0)),
        out_shape=jax.ShapeDtypeStruct((_NC, NP, D2b), jnp.float32),
    )(xp, agg1, W1a, b1a.reshape(1, H), W2a, b2a.reshape(1, H))

    srcs2 = jnp.stack([src3, src3 + NP])
    agg2 = _make_segsum(NP, E, D2b, 2)(
        hs.reshape(_NC * NP, D2b), srcs2, dsts, zacc).reshape(_NC, NP, D2b)

    h2s = pl.pallas_call(
        _mlp2_body,
        grid=(NP // R,),
        in_specs=[
            pl.BlockSpec((_NC, R, D2b), lambda i: (0, i, 0)),
            pl.BlockSpec((_NC, R, D2b), lambda i: (0, i, 0)),
            pl.BlockSpec((H, H), lambda i: (0, 0)),
            pl.BlockSpec((1, H), lambda i: (0, 0)),
            pl.BlockSpec((H, H), lambda i: (0, 0)),
            pl.BlockSpec((1, H), lambda i: (0, 0)),
        ],
        out_specs=pl.BlockSpec((_NC, R, D2b), lambda i: (0, i, 0)),
        out_shape=jax.ShapeDtypeStruct((_NC, NP, D2b), jnp.float32),
    )(hs, agg2, W1b, b1b.reshape(1, H), W2b, b2b.reshape(1, H))

    batchp = jnp.pad(batch, (0, NP - N), constant_values=_G)
    batch3 = batchp.reshape(_NS, NP // (_NS * _PCH), _PCH)
    zgd = jnp.zeros((_GP, D2b), jnp.float32)
    onesd = jnp.ones((_PCH, D2b), jnp.float32)
    sums, cnt = _make_pool(NP, D2b)(h2s.reshape(_NC * NP, D2b), batch3,
                                    zgd, onesd)

    out = pl.pallas_call(
        _final_body,
        in_specs=[
            pl.BlockSpec((_NC, _GP, D2b), lambda: (0, 0, 0)),
            pl.BlockSpec((_GP, D2b), lambda: (0, 0)),
            pl.BlockSpec((H, C), lambda: (0, 0)),
            pl.BlockSpec((1, C), lambda: (0, 0)),
        ],
        out_specs=pl.BlockSpec((_G, C), lambda: (0, 0)),
        out_shape=jax.ShapeDtypeStruct((_G, C), jnp.float32),
    )(sums, cnt, Wc, bc.reshape(1, C))
    return out
